# baseline (speedup 1.0000x reference)
import functools

import jax
import jax.numpy as jnp
from jax.experimental import pallas as pl
from jax.experimental.pallas import tpu as pltpu
from jax.experimental.pallas import tpu_sc as plsc

CURV_SCALE = 0.1
GATHER_WINDOW = 256
TC_BLOCK = 8192


def _gather_sc(table, idx_flat):
    n = idx_flat.shape[0]
    d = table.shape[1]
    mesh = plsc.VectorSubcoreMesh(core_axis_name="c", subcore_axis_name="s")
    idx2d = idx_flat.reshape(1, n)

    @functools.partial(
        pl.kernel,
        out_type=jax.ShapeDtypeStruct((n, d), table.dtype),
        mesh=mesh,
        compiler_params=pltpu.CompilerParams(use_tc_tiling_on_sc=False),
    )
    def gather_kernel(table_hbm, idx_hbm, out_hbm):
        def body(i_vmem, o_vmem):
            pltpu.sync_copy(table_hbm.at[i_vmem.at[0]], o_vmem)

        pltpu.emit_pipeline(
            body,
            grid=(n // GATHER_WINDOW,),
            in_specs=[pl.BlockSpec((1, GATHER_WINDOW), lambda i: (0, i))],
            out_specs=[pl.BlockSpec((GATHER_WINDOW, d), lambda i: (i, 0))],
            core_axis_name=("c", "s"),
            dimension_semantics=(pltpu.PARALLEL,),
        )(idx_hbm, out_hbm)

    return gather_kernel(table, idx2d)


def _project_tc(emb2d, w, b2d):
    n, d = emb2d.shape

    def body(x_ref, w_ref, b_ref, o_ref):
        raw = jnp.dot(x_ref[...], w_ref[...], preferred_element_type=jnp.float32)
        o_ref[...] = jnp.tanh(raw + b_ref[...]) * CURV_SCALE

    return pl.pallas_call(
        body,
        grid=(n // TC_BLOCK,),
        in_specs=[
            pl.BlockSpec((TC_BLOCK, d), lambda i: (i, 0)),
            pl.BlockSpec((d, d), lambda i: (0, 0)),
            pl.BlockSpec((1, d), lambda i: (0, 0)),
        ],
        out_specs=pl.BlockSpec((TC_BLOCK, d), lambda i: (i, 0)),
        out_shape=jax.ShapeDtypeStruct((n, d), jnp.float32),
    )(emb2d, w, b2d)


def kernel(token_ids, table, W_curv, b_curv):
    bsz, seq = token_ids.shape
    d = table.shape[1]
    idx = token_ids.reshape(-1).astype(jnp.int32)
    emb2d = _gather_sc(table, idx)
    curv2d = _project_tc(emb2d, W_curv, b_curv.reshape(1, -1))
    return emb2d.reshape(bsz, seq, d), curv2d.reshape(bsz, seq, d)


# speedup vs baseline: 1.7196x; 1.7196x over previous
import functools

import jax
import jax.numpy as jnp
from jax.experimental import pallas as pl
from jax.experimental.pallas import tpu as pltpu
from jax.experimental.pallas import tpu_sc as plsc

CURV_SCALE = 0.1
GATHER_WINDOW = 256


def _gather_sc(table, idx_flat):
    n = idx_flat.shape[0]
    d = table.shape[1]
    mesh = plsc.VectorSubcoreMesh(core_axis_name="c", subcore_axis_name="s")
    idx2d = idx_flat.reshape(1, n)

    @functools.partial(
        pl.kernel,
        out_type=jax.ShapeDtypeStruct((n, d), table.dtype),
        mesh=mesh,
        compiler_params=pltpu.CompilerParams(use_tc_tiling_on_sc=False),
    )
    def gather_kernel(table_hbm, idx_hbm, out_hbm):
        def body(i_vmem, o_vmem):
            pltpu.sync_copy(table_hbm.at[i_vmem.at[0]], o_vmem)

        pltpu.emit_pipeline(
            body,
            grid=(n // GATHER_WINDOW,),
            in_specs=[pl.BlockSpec((1, GATHER_WINDOW), lambda i: (0, i))],
            out_specs=[pl.BlockSpec((GATHER_WINDOW, d), lambda i: (i, 0))],
            core_axis_name=("c", "s"),
            dimension_semantics=(pltpu.PARALLEL,),
        )(idx_hbm, out_hbm)

    return gather_kernel(table, idx2d)


def _project_tc(emb4, w_t, b_col, seq, bsz, d):
    pack = 128 // d
    bq = bsz // pack

    def body(x_ref, w_ref, b_ref, e_ref, c_ref):
        x4 = x_ref[0]
        for c in range(pack):
            xc = x4[:, c * d:(c + 1) * d]
            x_t = xc.T
            e_ref[0, :, c * bq:(c + 1) * bq] = x_t
            raw = jnp.dot(w_ref[...], x_t, preferred_element_type=jnp.float32)
            c_ref[0, :, c * bq:(c + 1) * bq] = (
                jnp.tanh(raw + b_ref[...]) * CURV_SCALE
            )

    return pl.pallas_call(
        body,
        grid=(seq,),
        in_specs=[
            pl.BlockSpec((1, bsz // pack, 128), lambda i: (i, 0, 0)),
            pl.BlockSpec((d, d), lambda i: (0, 0)),
            pl.BlockSpec((d, 1), lambda i: (0, 0)),
        ],
        out_specs=[
            pl.BlockSpec((1, d, bsz), lambda i: (i, 0, 0)),
            pl.BlockSpec((1, d, bsz), lambda i: (i, 0, 0)),
        ],
        out_shape=[
            jax.ShapeDtypeStruct((seq, d, bsz), jnp.float32),
            jax.ShapeDtypeStruct((seq, d, bsz), jnp.float32),
        ],
    )(emb4, w_t, b_col)


def kernel(token_ids, table, W_curv, b_curv):
    bsz, seq = token_ids.shape
    d = table.shape[1]
    pack = 128 // d
    idx = (
        token_ids.T.astype(jnp.int32)
        .reshape(seq, pack, bsz // pack)
        .transpose(0, 2, 1)
        .reshape(-1)
    )
    emb_lin = _gather_sc(table, idx)
    emb4 = emb_lin.reshape(seq, (bsz * d) // 128, 128)
    emb_t, curv_t = _project_tc(
        emb4, W_curv.T, b_curv.reshape(d, 1), seq, bsz, d
    )
    return (
        jnp.transpose(emb_t, (2, 0, 1)),
        jnp.transpose(curv_t, (2, 0, 1)),
    )


# speedup vs baseline: 2.0239x; 1.1769x over previous
import functools

import jax
import jax.numpy as jnp
from jax.experimental import pallas as pl
from jax.experimental.pallas import tpu as pltpu
from jax.experimental.pallas import tpu_sc as plsc

CURV_SCALE = 0.1
GATHER_WINDOW = 256


def _gather_sc(table, idx_flat):
    n = idx_flat.shape[0]
    d = table.shape[1]
    mesh = plsc.VectorSubcoreMesh(core_axis_name="c", subcore_axis_name="s")
    idx2d = idx_flat.reshape(1, n)

    @functools.partial(
        pl.kernel,
        out_type=jax.ShapeDtypeStruct((n, d), table.dtype),
        mesh=mesh,
        compiler_params=pltpu.CompilerParams(use_tc_tiling_on_sc=False),
    )
    def gather_kernel(table_hbm, idx_hbm, out_hbm):
        def body(i_vmem, o_vmem):
            pltpu.sync_copy(table_hbm.at[i_vmem.at[0]], o_vmem)

        pltpu.emit_pipeline(
            body,
            grid=(n // GATHER_WINDOW,),
            in_specs=[pl.BlockSpec((1, GATHER_WINDOW), lambda i: (0, i))],
            out_specs=[pl.BlockSpec((GATHER_WINDOW, d), lambda i: (i, 0))],
            core_axis_name=("c", "s"),
            dimension_semantics=(pltpu.PARALLEL,),
        )(idx_hbm, out_hbm)

    return gather_kernel(table, idx2d)


def _pack_table_tc(tab_t, vocab, d):
    pack = 128 // d
    blk = 4096
    bq = blk // pack
    ng = -(-vocab // blk)
    rows_out = ng * bq

    def body(x_ref, o_ref):
        x = x_ref[...]
        z = jnp.concatenate(
            [x[:, a * bq:(a + 1) * bq] for a in range(pack)], axis=0
        )
        o_ref[...] = z.T

    return pl.pallas_call(
        body,
        grid=(ng,),
        in_specs=[pl.BlockSpec((d, blk), lambda i: (0, i))],
        out_specs=pl.BlockSpec((bq, 128), lambda i: (i, 0)),
        out_shape=jax.ShapeDtypeStruct((rows_out, 128), jnp.float32),
    )(tab_t)


def _project_tc(emb4, w_bd, b_col, seq, bsz, d):
    pack = 128 // d
    bq = bsz // pack

    def body(x_ref, w_ref, b_ref, e_ref, c_ref):
        x4t = x_ref[0].T
        raw = jnp.dot(w_ref[...], x4t, preferred_element_type=jnp.float32)
        y4 = jnp.tanh(raw + b_ref[...]) * CURV_SCALE
        for c in range(pack):
            e_ref[0, :, c * bq:(c + 1) * bq] = x4t[c * d:(c + 1) * d, :]
            c_ref[0, :, c * bq:(c + 1) * bq] = y4[c * d:(c + 1) * d, :]

    return pl.pallas_call(
        body,
        grid=(seq,),
        in_specs=[
            pl.BlockSpec((1, bq, 128), lambda i: (i, 0, 0)),
            pl.BlockSpec((pack * d, pack * d), lambda i: (0, 0)),
            pl.BlockSpec((pack * d, 1), lambda i: (0, 0)),
        ],
        out_specs=[
            pl.BlockSpec((1, d, bsz), lambda i: (i, 0, 0)),
            pl.BlockSpec((1, d, bsz), lambda i: (i, 0, 0)),
        ],
        out_shape=[
            jax.ShapeDtypeStruct((seq, d, bsz), jnp.float32),
            jax.ShapeDtypeStruct((seq, d, bsz), jnp.float32),
        ],
    )(emb4, w_bd, b_col)


def kernel(token_ids, table, W_curv, b_curv):
    bsz, seq = token_ids.shape
    vocab, d = table.shape
    pack = 128 // d
    idx = (
        token_ids.T.astype(jnp.int32)
        .reshape(seq, pack, bsz // pack)
        .transpose(0, 2, 1)
        .reshape(-1)
    )
    blk = 4096
    bq = blk // pack
    table4 = _pack_table_tc(table.T, vocab, d)
    u = idx % blk
    idx = pack * (bq * (idx // blk) + u % bq) + u // bq
    emb_lin = _gather_sc(table4.reshape(table4.shape[0] * pack, d), idx)
    emb4 = emb_lin.reshape(seq, (bsz * d) // 128, 128)
    w_bd = jnp.kron(jnp.eye(pack, dtype=W_curv.dtype), W_curv.T)
    b_col = jnp.tile(b_curv, pack).reshape(pack * d, 1)
    emb_t, curv_t = _project_tc(emb4, w_bd, b_col, seq, bsz, d)
    return (
        jnp.transpose(emb_t, (2, 0, 1)),
        jnp.transpose(curv_t, (2, 0, 1)),
    )
